# Initial kernel scaffold; baseline (speedup 1.0000x reference)
import jax
import jax.numpy as jnp
from jax.experimental import pallas as pl


def kernel(inputs, labels, labels_mask, unlabels_mask):
    raise NotImplementedError("write your pallas kernel here")


# speedup vs baseline: 11.4328x; 11.4328x over previous
import jax
import jax.numpy as jnp
from jax.experimental import pallas as pl
from jax.experimental.pallas import tpu as pltpu

_N = 10000
_D = 128
_C = 128
_N_LABELED = 1280
_K = int((_N - _N_LABELED) * 0.005)
_EPS = 1e-12
_INT_MIN = -(2 ** 31)
_B = 2000
_NB = _N // _B


def _rownorm(v):
    return v * jax.lax.rsqrt(jnp.maximum(jnp.sum(v * v, axis=1, keepdims=True), _EPS))


def _dot(a, b, ca, cb):
    return jax.lax.dot_general(
        a, b, (((ca,), (cb,)), ((), ())),
        preferred_element_type=jnp.float32,
        precision=jax.lax.Precision.HIGHEST,
    )


def _kth_mask(prob):
    int_min = jnp.int32(_INT_MIN)
    i32 = jax.lax.bitcast_convert_type(prob, jnp.int32)
    s = jnp.where(i32 >= 0, i32, jnp.bitwise_or(jnp.bitwise_not(i32), int_min))
    t = jnp.full((prob.shape[0], 1), _INT_MIN, jnp.int32)
    for b in range(31, -1, -1):
        inc = int_min if b == 31 else jnp.int32(1 << b)
        cand = t + inc
        cnt = jnp.sum((s >= cand).astype(jnp.int32), axis=1, keepdims=True)
        t = jnp.where(cnt >= _K, cand, t)
    return jnp.where(s >= t, prob, 0.0)


def _proto_body(x_ref, lab_ref, tot_ref, cnt_ref):
    i = pl.program_id(0)

    @pl.when(i == 0)
    def _():
        tot_ref[...] = jnp.zeros_like(tot_ref)
        cnt_ref[...] = jnp.zeros_like(cnt_ref)

    x = x_ref[...]
    lab = lab_ref[...]
    tot_ref[...] += _dot(lab, x, 0, 0)
    cnt_ref[...] += _dot(lab, jnp.ones((_B, 1), jnp.float32), 0, 0)


def _topk_body(x_ref, um_ref, tot_ref, cnt_ref, pn2_ref, supx_ref, colup_ref,
               pn_ref, protos_ref):
    i = pl.program_id(0)

    @pl.when(i == 0)
    def _():
        countc = cnt_ref[...]
        protos = tot_ref[...] * (1.0 / countc)
        protos_ref[...] = protos
        pn_ref[...] = _rownorm(protos)
        supx_ref[...] = jnp.zeros_like(supx_ref)
        colup_ref[...] = jnp.zeros_like(colup_ref)

    x = x_ref[...]
    xn = _rownorm(x)
    prob = _dot(xn, pn_ref[...], 1, 1)
    up = _kth_mask(prob) * um_ref[...]
    supx_ref[...] += _dot(up, x, 0, 0)
    colup_ref[...] += _dot(up, jnp.ones((_B, 1), jnp.float32), 0, 0)

    @pl.when(i == _NB - 1)
    def _():
        protos = protos_ref[...]
        colup = colup_ref[...]
        denom = colup + cnt_ref[...]
        protos2 = protos + (supx_ref[...] - protos * colup) / denom
        pn2_ref[...] = _rownorm(protos2)


def _logits_body(x_ref, pn2_ref, out_ref):
    out_ref[...] = _dot(_rownorm(x_ref[...]), pn2_ref[...], 1, 1)


def _row_spec():
    return pl.BlockSpec((_B, _D), lambda i: (i, 0))


def _const_spec(r):
    return pl.BlockSpec((_C, r), lambda i: (0, 0))


def kernel(inputs, labels, labels_mask, unlabels_mask):
    del labels_mask
    f32 = jnp.float32
    um = unlabels_mask.astype(f32).reshape(_N, 1)
    x = inputs
    lab = labels.astype(f32)

    tot, cnt = pl.pallas_call(
        _proto_body,
        grid=(_NB,),
        in_specs=[_row_spec(), _row_spec()],
        out_specs=[_const_spec(_D), _const_spec(1)],
        out_shape=[jax.ShapeDtypeStruct((_C, _D), f32),
                   jax.ShapeDtypeStruct((_C, 1), f32)],
    )(x, lab)

    pn2, _supx, _colup = pl.pallas_call(
        _topk_body,
        grid=(_NB,),
        in_specs=[_row_spec(), pl.BlockSpec((_B, 1), lambda i: (i, 0)),
                  _const_spec(_D), _const_spec(1)],
        out_specs=[_const_spec(_D), _const_spec(_D), _const_spec(1)],
        out_shape=[jax.ShapeDtypeStruct((_C, _D), f32),
                   jax.ShapeDtypeStruct((_C, _D), f32),
                   jax.ShapeDtypeStruct((_C, 1), f32)],
        scratch_shapes=[pltpu.VMEM((_C, _D), f32), pltpu.VMEM((_C, _D), f32)],
    )(x, um, tot, cnt)

    return pl.pallas_call(
        _logits_body,
        grid=(_NB,),
        in_specs=[_row_spec(), _const_spec(_D)],
        out_specs=pl.BlockSpec((_B, _C), lambda i: (i, 0)),
        out_shape=jax.ShapeDtypeStruct((_N, _C), f32),
    )(x, pn2)


# speedup vs baseline: 24.1156x; 2.1093x over previous
import jax
import jax.numpy as jnp
from jax.experimental import pallas as pl
from jax.experimental.pallas import tpu as pltpu

_N = 10000
_D = 128
_C = 128
_N_LABELED = 1280
_K = int((_N - _N_LABELED) * 0.005)
_EPS = 1e-12
_INT_MIN = -(2 ** 31)
_B = 2000
_NB = _N // _B


def _rownorm(v):
    return v * jax.lax.rsqrt(jnp.maximum(jnp.sum(v * v, axis=1, keepdims=True), _EPS))


def _dot(a, b, ca, cb):
    return jax.lax.dot_general(
        a, b, (((ca,), (cb,)), ((), ())),
        preferred_element_type=jnp.float32,
        precision=jax.lax.Precision.HIGHEST,
    )


def _kth_mask_t(probt):
    int_min = jnp.int32(_INT_MIN)
    i32 = jax.lax.bitcast_convert_type(probt, jnp.int32)
    s = jnp.where(i32 >= 0, i32, jnp.bitwise_or(jnp.bitwise_not(i32), int_min))
    t = jnp.full((1, probt.shape[1]), _INT_MIN, jnp.int32)
    for b in range(31, -1, -1):
        inc = int_min if b == 31 else jnp.int32(1 << b)
        cand = t + inc
        cnt = jnp.sum((s >= cand).astype(jnp.int32), axis=0, keepdims=True)
        t = jnp.where(cnt >= _K, cand, t)
    return jnp.where(s >= t, probt, 0.0)


def _proto_body(x_ref, lab_ref, tot_ref, cnt_ref):
    i = pl.program_id(0)

    @pl.when(i == 0)
    def _():
        tot_ref[...] = jnp.zeros_like(tot_ref)
        cnt_ref[...] = jnp.zeros_like(cnt_ref)

    x = x_ref[...]
    lab = lab_ref[...]
    tot_ref[...] += _dot(lab, x, 0, 0)
    cnt_ref[...] += _dot(lab, jnp.ones((_B, 1), jnp.float32), 0, 0)


def _topk_body(x_ref, um_ref, tot_ref, cnt_ref, pn2_ref, supx_ref, colup_ref,
               pn_ref, protos_ref):
    i = pl.program_id(0)

    @pl.when(i == 0)
    def _():
        countc = cnt_ref[...]
        protos = tot_ref[...] * (1.0 / countc)
        protos_ref[...] = protos
        pn_ref[...] = _rownorm(protos)
        supx_ref[...] = jnp.zeros_like(supx_ref)
        colup_ref[...] = jnp.zeros_like(colup_ref)

    x = x_ref[...]
    xn = _rownorm(x)
    probt = _dot(pn_ref[...], xn, 1, 1)
    upt = _kth_mask_t(probt) * um_ref[0]
    supx_ref[...] += _dot(upt, x, 1, 0)
    colup_ref[...] += _dot(upt, jnp.ones((_B, 1), jnp.float32), 1, 0)

    @pl.when(i == _NB - 1)
    def _():
        protos = protos_ref[...]
        colup = colup_ref[...]
        denom = colup + cnt_ref[...]
        protos2 = protos + (supx_ref[...] - protos * colup) / denom
        pn2_ref[...] = _rownorm(protos2)


def _logits_body(x_ref, pn2_ref, out_ref):
    out_ref[...] = _dot(_rownorm(x_ref[...]), pn2_ref[...], 1, 1)


def _row_spec():
    return pl.BlockSpec((_B, _D), lambda i: (i, 0))


def _const_spec(r):
    return pl.BlockSpec((_C, r), lambda i: (0, 0))


def kernel(inputs, labels, labels_mask, unlabels_mask):
    del labels_mask
    f32 = jnp.float32
    um = unlabels_mask.astype(f32).reshape(_NB, 1, _B)
    x = inputs
    lab = labels.astype(f32)

    tot, cnt = pl.pallas_call(
        _proto_body,
        grid=(_NB,),
        in_specs=[_row_spec(), _row_spec()],
        out_specs=[_const_spec(_D), _const_spec(1)],
        out_shape=[jax.ShapeDtypeStruct((_C, _D), f32),
                   jax.ShapeDtypeStruct((_C, 1), f32)],
    )(x, lab)

    pn2, _supx, _colup = pl.pallas_call(
        _topk_body,
        grid=(_NB,),
        in_specs=[_row_spec(), pl.BlockSpec((1, 1, _B), lambda i: (i, 0, 0)),
                  _const_spec(_D), _const_spec(1)],
        out_specs=[_const_spec(_D), _const_spec(_D), _const_spec(1)],
        out_shape=[jax.ShapeDtypeStruct((_C, _D), f32),
                   jax.ShapeDtypeStruct((_C, _D), f32),
                   jax.ShapeDtypeStruct((_C, 1), f32)],
        scratch_shapes=[pltpu.VMEM((_C, _D), f32), pltpu.VMEM((_C, _D), f32)],
    )(x, um, tot, cnt)

    return pl.pallas_call(
        _logits_body,
        grid=(_NB,),
        in_specs=[_row_spec(), _const_spec(_D)],
        out_specs=pl.BlockSpec((_B, _C), lambda i: (i, 0)),
        out_shape=jax.ShapeDtypeStruct((_N, _C), f32),
    )(x, pn2)
